# baseline (speedup 1.0000x reference)
import functools

import jax
import jax.numpy as jnp
from jax import lax
from jax.experimental import pallas as pl
from jax.experimental.pallas import tpu as pltpu
from jax.experimental.pallas import tpu_sc as plsc

VOCAB = 1000
HIDDEN = 128
B = 1024
L = 50
NC = 2
NS = 16
NW = NC * NS
TOKENS = B * L
TOK_PER_W = TOKENS // NW
B_PER_W = B // NW
CHUNK = 40
NCHUNK = TOK_PER_W // CHUNK


def _fuse_kernel(emb_ref, pw_ref, pb_ref, rw_ref, rb_ref, m_ref, r8_ref):
    emb = emb_ref[...]
    m_ref[...] = (
        jnp.dot(emb, pw_ref[...], preferred_element_type=jnp.float32)
        + pb_ref[...]
    )
    r = (
        jnp.dot(emb, rw_ref[...], preferred_element_type=jnp.float32)
        + rb_ref[...]
    )
    r8_ref[...] = jnp.broadcast_to(r, (VOCAB, 16))


def _fuse(embed_table, proj_w, proj_b, reg_w, reg_b):
    return pl.pallas_call(
        _fuse_kernel,
        out_shape=(
            jax.ShapeDtypeStruct((VOCAB, VOCAB), jnp.float32),
            jax.ShapeDtypeStruct((VOCAB, 16), jnp.float32),
        ),
    )(embed_table, proj_w, proj_b.reshape(1, VOCAB), reg_w,
      reg_b.reshape(1, 1))


def _sc_body(m_hbm, r8_hbm, ids_hbm, logits_hbm, pred_hbm,
             idx_v, buf_v, rbuf_v, pred_v, acc_s,
             gsem0, gsem1, rsem0, rsem1, ssem0, ssem1):
    wid = lax.axis_index("c") * NS + lax.axis_index("s")
    tok0 = wid * TOK_PER_W
    gsems = (gsem0, gsem1)
    rsems = (rsem0, rsem1)
    ssems = (ssem0, ssem1)

    pltpu.sync_copy(ids_hbm.at[pl.ds(tok0, TOK_PER_W)], idx_v)
    for b in range(B_PER_W):
        acc_s[b] = jnp.float32(0.0)

    def idx_slice(c):
        return idx_v.at[pl.ds(c * CHUNK, CHUNK)]

    def gather(c, par):
        pltpu.async_copy(m_hbm.at[idx_slice(c)], buf_v.at[par], gsems[par])
        pltpu.async_copy(r8_hbm.at[idx_slice(c)], rbuf_v.at[par],
                         rsems[par])

    def store_desc(c, par):
        return pltpu.make_async_copy(
            buf_v.at[par], logits_hbm.at[pl.ds(tok0 + c * CHUNK, CHUNK)],
            ssems[par])

    gather(0, 0)

    def pair_step(g, carry):
        del carry
        for par in (0, 1):
            c = g * 2 + par
            if par == 0:
                @pl.when(c >= 1)
                def _():
                    store_desc(c - 1, 1 - par).wait()
            else:
                store_desc(c - 1, 1 - par).wait()

            @pl.when(c + 1 < NCHUNK)
            def _():
                gather(c + 1, 1 - par)
            pltpu.make_async_copy(
                m_hbm.at[idx_slice(c)], buf_v.at[par], gsems[par]).wait()
            store_desc(c, par).start()
            pltpu.make_async_copy(
                r8_hbm.at[idx_slice(c)], rbuf_v.at[par], rsems[par]).wait()

            b_first = (c * CHUNK) // L
            b_last = (c * CHUNK + CHUNK - 1) // L
            split = jnp.minimum(L * (b_first + 1) - c * CHUNK, CHUNK)

            def tok_step(i, carry2):
                a, bb = carry2
                row = rbuf_v[par, i, :]
                zero = jnp.zeros((16,), jnp.float32)
                a = a + jnp.where(i < split, row, zero)
                bb = bb + jnp.where(i >= split, row, zero)
                return (a, bb)

            zeros16 = jnp.zeros((16,), jnp.float32)
            a, bb = lax.fori_loop(0, CHUNK, tok_step, (zeros16, zeros16))
            acc_s[b_first] = acc_s[b_first] + a[0]
            acc_s[b_last] = acc_s[b_last] + bb[0]
        return 0

    lax.fori_loop(0, NCHUNK // 2, pair_step, 0)
    store_desc(NCHUNK - 1, 1).wait()

    lane = lax.iota(jnp.int32, 16)
    for k in (0, 1):
        acc = jnp.zeros((16,), jnp.float32)
        for i in range(16):
            v = acc_s[k * 16 + i] * jnp.float32(1.0 / L)
            acc = jnp.where(lane == i, jnp.broadcast_to(v, (16,)), acc)
        pred_v[pl.ds(k * 16, 16)] = acc
    pltpu.sync_copy(pred_v, pred_hbm.at[pl.ds(wid * B_PER_W, B_PER_W)])


@functools.partial(
    pl.kernel,
    mesh=plsc.VectorSubcoreMesh(core_axis_name="c", subcore_axis_name="s"),
    compiler_params=pltpu.CompilerParams(use_tc_tiling_on_sc=False),
    out_type=(
        jax.ShapeDtypeStruct((TOKENS, VOCAB), jnp.float32),
        jax.ShapeDtypeStruct((B,), jnp.float32),
    ),
    scratch_types=[
        pltpu.VMEM((TOK_PER_W,), jnp.int32),
        pltpu.VMEM((2, CHUNK, VOCAB), jnp.float32),
        pltpu.VMEM((2, CHUNK, 16), jnp.float32),
        pltpu.VMEM((B_PER_W,), jnp.float32),
        pltpu.SMEM((B_PER_W,), jnp.float32),
        pltpu.SemaphoreType.DMA,
        pltpu.SemaphoreType.DMA,
        pltpu.SemaphoreType.DMA,
        pltpu.SemaphoreType.DMA,
        pltpu.SemaphoreType.DMA,
        pltpu.SemaphoreType.DMA,
    ],
)
def _sc_gather(m_hbm, r8_hbm, ids_hbm, logits_hbm, pred_hbm,
               idx_v, buf_v, rbuf_v, pred_v, acc_s,
               gsem0, gsem1, rsem0, rsem1, ssem0, ssem1):
    _sc_body(m_hbm, r8_hbm, ids_hbm, logits_hbm, pred_hbm,
             idx_v, buf_v, rbuf_v, pred_v, acc_s,
             gsem0, gsem1, rsem0, rsem1, ssem0, ssem1)


def kernel(input_ids, embed_table, proj_w, proj_b, reg_w, reg_b):
    m, r8 = _fuse(embed_table, proj_w, proj_b, reg_w, reg_b)
    ids_flat = input_ids.reshape(TOKENS).astype(jnp.int32)
    logits_flat, pred = _sc_gather(m, r8, ids_flat)
    return logits_flat.reshape(B, L, VOCAB), pred.reshape(B, 1)


# speedup vs baseline: 1.3900x; 1.3900x over previous
import functools

import jax
import jax.numpy as jnp
from jax import lax
from jax.experimental import pallas as pl
from jax.experimental.pallas import tpu as pltpu
from jax.experimental.pallas import tpu_sc as plsc

VOCAB = 1000
HIDDEN = 128
B = 1024
L = 50
NC = 2
NS = 16
NW = NC * NS
TOKENS = B * L
TOK_PER_W = TOKENS // NW
B_PER_W = B // NW
TILE = 512
NT = TOKENS // TILE


def _fuse_r8_kernel(emb_ref, rw_ref, rb_ref, r8_ref):
    r = (
        jnp.dot(emb_ref[...], rw_ref[...], preferred_element_type=jnp.float32)
        + rb_ref[...]
    )
    r8_ref[...] = jnp.broadcast_to(r, (VOCAB, 16))


def _fuse_r8(embed_table, reg_w, reg_b):
    return pl.pallas_call(
        _fuse_r8_kernel,
        out_shape=jax.ShapeDtypeStruct((VOCAB, 16), jnp.float32),
    )(embed_table, reg_w, reg_b.reshape(1, 1))


def _logits_kernel(ids_ref, emb_ref, pw_ref, pb_ref, out_ref):
    ids = ids_ref[0]
    v = lax.broadcasted_iota(jnp.int32, (TILE, VOCAB), 1)
    oh = (ids == v).astype(jnp.float32)
    hid = jnp.dot(oh, emb_ref[...], preferred_element_type=jnp.float32)
    out_ref[...] = (
        jnp.dot(hid, pw_ref[...], preferred_element_type=jnp.float32)
        + pb_ref[...]
    )


def _logits(ids3, embed_table, proj_w, proj_b2):
    return pl.pallas_call(
        _logits_kernel,
        grid=(NT,),
        in_specs=[
            pl.BlockSpec((1, TILE, 1), lambda i: (i, 0, 0)),
            pl.BlockSpec((VOCAB, HIDDEN), lambda i: (0, 0)),
            pl.BlockSpec((HIDDEN, VOCAB), lambda i: (0, 0)),
            pl.BlockSpec((1, VOCAB), lambda i: (0, 0)),
        ],
        out_specs=pl.BlockSpec((TILE, VOCAB), lambda i: (i, 0)),
        out_shape=jax.ShapeDtypeStruct((TOKENS, VOCAB), jnp.float32),
    )(ids3, embed_table, proj_w, proj_b2)


CB = 8
CTOK = CB * L
NCH = B_PER_W // CB


def _sc_pred_body(r8_hbm, ids_hbm, pred_hbm,
                  idx_v, buf_v, pred_v, acc_s, gsem0, gsem1):
    wid = lax.axis_index("c") * NS + lax.axis_index("s")
    tok0 = wid * TOK_PER_W
    gsems = (gsem0, gsem1)

    pltpu.sync_copy(ids_hbm.at[pl.ds(tok0, TOK_PER_W)], idx_v)

    def idx_slice(c):
        return idx_v.at[pl.ds(c * CTOK, CTOK)]

    def gather(c, par):
        pltpu.async_copy(r8_hbm.at[idx_slice(c)], buf_v.at[par], gsems[par])

    gather(0, 0)

    def pair_step(g, carry):
        del carry
        for par in (0, 1):
            c = g * 2 + par

            @pl.when(c + 1 < NCH)
            def _():
                gather(c + 1, 1 - par)

            pltpu.make_async_copy(
                r8_hbm.at[idx_slice(c)], buf_v.at[par], gsems[par]).wait()

            for bb in range(CB):
                def tok_step(i, acc, bb=bb):
                    return acc + buf_v[par, bb * L + i, :]

                acc = lax.fori_loop(0, L, tok_step,
                                    jnp.zeros((16,), jnp.float32))
                acc_s[c * CB + bb] = acc[0] * jnp.float32(1.0 / L)
        return 0

    lax.fori_loop(0, NCH // 2, pair_step, 0)

    lane = lax.iota(jnp.int32, 16)
    for k in (0, 1):
        acc = jnp.zeros((16,), jnp.float32)
        for i in range(16):
            v = acc_s[k * 16 + i]
            acc = jnp.where(lane == i, jnp.broadcast_to(v, (16,)), acc)
        pred_v[pl.ds(k * 16, 16)] = acc
    pltpu.sync_copy(pred_v, pred_hbm.at[pl.ds(wid * B_PER_W, B_PER_W)])


@functools.partial(
    pl.kernel,
    mesh=plsc.VectorSubcoreMesh(core_axis_name="c", subcore_axis_name="s"),
    compiler_params=pltpu.CompilerParams(use_tc_tiling_on_sc=False),
    out_type=jax.ShapeDtypeStruct((B,), jnp.float32),
    scratch_types=[
        pltpu.VMEM((TOK_PER_W,), jnp.int32),
        pltpu.VMEM((2, CTOK, 16), jnp.float32),
        pltpu.VMEM((B_PER_W,), jnp.float32),
        pltpu.SMEM((B_PER_W,), jnp.float32),
        pltpu.SemaphoreType.DMA,
        pltpu.SemaphoreType.DMA,
    ],
)
def _sc_pred(r8_hbm, ids_hbm, pred_hbm,
             idx_v, buf_v, pred_v, acc_s, gsem0, gsem1):
    _sc_pred_body(r8_hbm, ids_hbm, pred_hbm,
                  idx_v, buf_v, pred_v, acc_s, gsem0, gsem1)


def kernel(input_ids, embed_table, proj_w, proj_b, reg_w, reg_b):
    ids_flat = input_ids.reshape(TOKENS).astype(jnp.int32)
    r8 = _fuse_r8(embed_table, reg_w, reg_b)
    pred = _sc_pred(r8, ids_flat)
    logits_flat = _logits(
        ids_flat.reshape(NT, TILE, 1), embed_table, proj_w,
        proj_b.reshape(1, VOCAB))
    return logits_flat.reshape(B, L, VOCAB), pred.reshape(B, 1)


# speedup vs baseline: 1.7566x; 1.2637x over previous
import functools

import jax
import jax.numpy as jnp
from jax import lax
from jax.experimental import pallas as pl
from jax.experimental.pallas import tpu as pltpu
from jax.experimental.pallas import tpu_sc as plsc

VOCAB = 1000
HIDDEN = 128
B = 1024
L = 50
NC = 2
NS = 16
NW = NC * NS
TOKENS = B * L
TOK_PER_W = TOKENS // NW
B_PER_W = B // NW


def _fuse_r8_kernel(emb_ref, rw_ref, rb_ref, r8_ref):
    r = (
        jnp.dot(emb_ref[...], rw_ref[...], preferred_element_type=jnp.float32)
        + rb_ref[...]
    )
    r8_ref[...] = jnp.broadcast_to(r, (VOCAB, 16))


def _fuse_r8(embed_table, reg_w, reg_b):
    return pl.pallas_call(
        _fuse_r8_kernel,
        out_shape=jax.ShapeDtypeStruct((VOCAB, 16), jnp.float32),
    )(embed_table, reg_w, reg_b.reshape(1, 1))


TB = 8
NT = B // TB


def _logits_kernel(ids_ref, emb_ref, pw_ref, pb_ref, out_ref):
    ids = ids_ref[...][:, :, None]
    v = lax.broadcasted_iota(jnp.int32, (TB, L, VOCAB), 2)
    oh = (ids == v).astype(jnp.float32)
    hid = lax.dot_general(
        oh, emb_ref[...], (((2,), (0,)), ((), ())),
        preferred_element_type=jnp.float32)
    out_ref[...] = lax.dot_general(
        hid, pw_ref[...], (((2,), (0,)), ((), ())),
        preferred_element_type=jnp.float32) + pb_ref[...]


def _logits(ids2, embed_table, proj_w, proj_b3):
    return pl.pallas_call(
        _logits_kernel,
        grid=(NT,),
        in_specs=[
            pl.BlockSpec((TB, L), lambda i: (i, 0)),
            pl.BlockSpec((VOCAB, HIDDEN), lambda i: (0, 0)),
            pl.BlockSpec((HIDDEN, VOCAB), lambda i: (0, 0)),
            pl.BlockSpec((1, 1, VOCAB), lambda i: (0, 0, 0)),
        ],
        out_specs=pl.BlockSpec((TB, L, VOCAB), lambda i: (i, 0, 0)),
        out_shape=jax.ShapeDtypeStruct((B, L, VOCAB), jnp.float32),
    )(ids2, embed_table, proj_w, proj_b3)


CB = 8
CTOK = CB * L
NCH = B_PER_W // CB


def _sc_pred_body(r8_hbm, ids_hbm, pred_hbm,
                  idx_v, buf_v, pred_v, acc_s, gsem0, gsem1):
    wid = lax.axis_index("c") * NS + lax.axis_index("s")
    tok0 = wid * TOK_PER_W
    gsems = (gsem0, gsem1)

    pltpu.sync_copy(ids_hbm.at[pl.ds(tok0, TOK_PER_W)], idx_v)

    def idx_slice(c):
        return idx_v.at[pl.ds(c * CTOK, CTOK)]

    def gather(c, par):
        pltpu.async_copy(r8_hbm.at[idx_slice(c)], buf_v.at[par], gsems[par])

    gather(0, 0)

    def pair_step(g, carry):
        del carry
        for par in (0, 1):
            c = g * 2 + par

            @pl.when(c + 1 < NCH)
            def _():
                gather(c + 1, 1 - par)

            pltpu.make_async_copy(
                r8_hbm.at[idx_slice(c)], buf_v.at[par], gsems[par]).wait()

            for bb in range(CB):
                def tok_step(i, acc, bb=bb):
                    return acc + buf_v[par, bb * L + i, :]

                acc = lax.fori_loop(0, L, tok_step,
                                    jnp.zeros((16,), jnp.float32))
                acc_s[c * CB + bb] = acc[0] * jnp.float32(1.0 / L)
        return 0

    lax.fori_loop(0, NCH // 2, pair_step, 0)

    lane = lax.iota(jnp.int32, 16)
    for k in (0, 1):
        acc = jnp.zeros((16,), jnp.float32)
        for i in range(16):
            v = acc_s[k * 16 + i]
            acc = jnp.where(lane == i, jnp.broadcast_to(v, (16,)), acc)
        pred_v[pl.ds(k * 16, 16)] = acc
    pltpu.sync_copy(pred_v, pred_hbm.at[pl.ds(wid * B_PER_W, B_PER_W)])


@functools.partial(
    pl.kernel,
    mesh=plsc.VectorSubcoreMesh(core_axis_name="c", subcore_axis_name="s"),
    compiler_params=pltpu.CompilerParams(use_tc_tiling_on_sc=False),
    out_type=jax.ShapeDtypeStruct((B,), jnp.float32),
    scratch_types=[
        pltpu.VMEM((TOK_PER_W,), jnp.int32),
        pltpu.VMEM((2, CTOK, 16), jnp.float32),
        pltpu.VMEM((B_PER_W,), jnp.float32),
        pltpu.SMEM((B_PER_W,), jnp.float32),
        pltpu.SemaphoreType.DMA,
        pltpu.SemaphoreType.DMA,
    ],
)
def _sc_pred(r8_hbm, ids_hbm, pred_hbm,
             idx_v, buf_v, pred_v, acc_s, gsem0, gsem1):
    _sc_pred_body(r8_hbm, ids_hbm, pred_hbm,
                  idx_v, buf_v, pred_v, acc_s, gsem0, gsem1)


def kernel(input_ids, embed_table, proj_w, proj_b, reg_w, reg_b):
    ids2 = input_ids.astype(jnp.int32)
    r8 = _fuse_r8(embed_table, reg_w, reg_b)
    pred = _sc_pred(r8, ids2.reshape(TOKENS))
    logits = _logits(ids2, embed_table, proj_w,
                     proj_b.reshape(1, 1, VOCAB))
    return logits, pred.reshape(B, 1)
